# baseline (speedup 1.0000x reference)
import jax
import jax.numpy as jnp
from jax import lax
from jax.experimental import pallas as pl
from jax.experimental.pallas import tpu as pltpu
from jax.experimental.pallas import tpu_sc as plsc

N = 10000
NPAD = 10240
H = 256
R = 4
E = 40000
NS = 16
NC = 2
CH = 128
NCH = 20
EPT = CH * NCH
EREAL = E // NS
STRIPE = NPAD // NS
WBR = 80


def _pieces(dout):
    pw = min(64, dout // NC)
    return dout // pw, pw


def _tc_dense(h_parts, W, al, ar, dout, np_in, pw_in, bias_prev):
    NP, PW = _pieces(dout)
    bm = 1024
    MB = NPAD // bm

    def _dot3(a, b):
        ah = a.astype(jnp.bfloat16)
        alo = (a - ah.astype(jnp.float32)).astype(jnp.bfloat16)
        bh = b.astype(jnp.bfloat16)
        blo = (b - bh.astype(jnp.float32)).astype(jnp.bfloat16)
        f = jnp.float32
        return (jnp.dot(ah, bh, preferred_element_type=f)
                + jnp.dot(ah, blo, preferred_element_type=f)
                + jnp.dot(alo, bh, preferred_element_type=f))

    def body(h_ref, w_ref, al_ref, ar_ref, *rest):
        if bias_prev is not None:
            b_ref, z_ref, el_ref, er_ref = rest
        else:
            z_ref, el_ref, er_ref = rest
        w = w_ref[0]
        if np_in == 1:
            hb = h_ref[...]
            z = _dot3(hb, w)
        else:
            z = jnp.zeros((bm, dout), jnp.float32)
            for i in range(np_in):
                hp = h_ref[i]
                if bias_prev is not None:
                    hp = jnp.maximum(hp + b_ref[i], 0.0)
                z = z + _dot3(hp, w[i * pw_in:(i + 1) * pw_in, :])
        if dout == 2 * H // 2:
            for ch in range(NC):
                z_ref[ch] = z[:, ch * 128:(ch + 1) * 128]
        else:
            for p in range(NP):
                z_ref[p] = z[:, p * PW:(p + 1) * PW]
        el_ref[0, 0] = jnp.dot(z, al_ref[0, 0], preferred_element_type=jnp.float32)
        er_ref[0, 0] = jnp.dot(z, ar_ref[0, 0], preferred_element_type=jnp.float32)

    if np_in == 1:
        h_spec = pl.BlockSpec((bm, H), lambda m, r: (m, 0))
    else:
        h_spec = pl.BlockSpec((np_in, bm, pw_in), lambda m, r: (0, m, 0))
    in_specs = [
            h_spec,
            pl.BlockSpec((1, H, dout), lambda m, r: (r, 0, 0)),
            pl.BlockSpec((1, 1, dout), lambda m, r: (r, 0, 0)),
            pl.BlockSpec((1, 1, dout), lambda m, r: (r, 0, 0)),
    ]
    args = [h_parts, W, al.reshape(R, 1, dout), ar.reshape(R, 1, dout)]
    if bias_prev is not None:
        in_specs.append(
            pl.BlockSpec((np_in, 1, pw_in), lambda m, r: (0, 0, 0)))
        args.append(bias_prev.reshape(np_in, 1, pw_in))
    zf, el, er = pl.pallas_call(
        body,
        grid=(MB, R),
        in_specs=in_specs,
        out_specs=[
            (pl.BlockSpec((NC, bm, 128), lambda m, r: (r, m, 0))
             if dout == 256 else
             pl.BlockSpec((NP, bm, PW), lambda m, r: (r, m, 0))),
            pl.BlockSpec((1, 1, bm), lambda m, r: (r, 0, m)),
            pl.BlockSpec((1, 1, bm), lambda m, r: (r, 0, m)),
        ],
        out_shape=[
            (jax.ShapeDtypeStruct((R * NC, NPAD, 128), jnp.float32)
             if dout == 256 else
             jax.ShapeDtypeStruct((R * NP, NPAD, PW), jnp.float32)),
            jax.ShapeDtypeStruct((R, 1, NPAD), jnp.float32),
            jax.ShapeDtypeStruct((R, 1, NPAD), jnp.float32),
        ],
    )(*args)
    return (zf.reshape(R * NP * NPAD, PW), el.reshape(R, NPAD),
            er.reshape(R, NPAD))


def _sc_edge(Zf, el, er, src3, dst3, bsum, dout, relu, add_bias):
    NP, PW = _pieces(dout)
    NPQ = NP // NC
    KD = PW // 16
    f32 = jnp.float32

    NB = 2
    NGR = NCH // NB

    def body(zf_hbm, el_hbm, er_hbm, src_hbm, dst_hbm, bsum_hbm, out_hbm,
             srcb, dstb, exb, elb, erb, invb, dstripe, idxb, abuf, gbufs,
             wbuf, bbuf, acc, den0, den1, den2, den3,
             gsem0, gsem1, ssem0, ssem1,
             dsem0, dsem1):
        c = lax.axis_index("c").astype(jnp.int32)
        s = lax.axis_index("s").astype(jnp.int32)
        i32 = jnp.int32
        dens = [den0, den1, den2, den3]
        gsems = [gsem0, gsem1]
        ssems = [ssem0, ssem1]
        dsems = [dsem0, dsem1]
        zero16 = jnp.zeros((16,), f32)

        def drain_scatter(b):
            pltpu.make_async_copy(
                zf_hbm.at[pl.ds(0, CH)], gbufs.at[b], ssems[b]).wait()

        def drain_den(b):
            pltpu.make_async_copy(
                el_hbm.at[0, pl.ds(0, CH)], abuf, dsems[b]).wait()

        pltpu.sync_copy(src_hbm.at[pl.ds(s * i32(R * NCH), R * NCH)], srcb)
        pltpu.sync_copy(dst_hbm.at[pl.ds(s * i32(R * NCH), R * NCH)], dstb)

        @pl.loop(0, WBR)
        def _zw(i):
            for k in range(KD):
                wbuf[i, pl.ds(16 * k, 16)] = zero16
        for k8 in range(STRIPE // WBR):
            pltpu.sync_copy(wbuf, acc.at[pl.ds(s * i32(STRIPE) + i32(k8 * WBR), WBR)])
        for k in range(8):
            abuf[pl.ds(16 * k, 16)] = zero16
        for r in range(R):
            for k5 in range(STRIPE // CH):
                pltpu.sync_copy(abuf, dens[r].at[pl.ds(s * i32(STRIPE) + i32(k5 * CH), CH)])
        plsc.subcore_barrier()

        for r in range(R):
            pltpu.sync_copy(el_hbm.at[r], elb)
            pltpu.sync_copy(er_hbm.at[r], erb)

            @pl.loop(0, NCH // 2)
            def _p1(g, r=r):
                for b in range(2):
                    j = g * i32(2) + i32(b)
                    row = j + i32(r * NCH)
                    for k in range(8):
                        sv = srcb[row, pl.ds(16 * k, 16)]
                        dv = dstb[row, pl.ds(16 * k, 16)]
                        ev = (plsc.load_gather(elb, [sv])
                              + plsc.load_gather(erb, [dv]))
                        ev = jnp.where(ev > 0, ev, 0.2 * ev)
                        ex = jnp.exp(ev)
                        pos = j * i32(CH) + i32(16 * k) + lax.iota(jnp.int32, 16)
                        ex = jnp.where(pos < EREAL, ex, 0.0)
                        exb[row, pl.ds(16 * k, 16)] = ex

                    @pl.when(g > 0)
                    def _dr(b=b):
                        drain_den(b)
                    pltpu.async_copy(exb.at[row], dens[r].at[dstb.at[row]],
                                     dsems[b], add=True)
            for b in range(2):
                drain_den(b)
        plsc.subcore_barrier()

        for r in range(R):
            pltpu.sync_copy(dens[r].at[pl.ds(s * i32(STRIPE), STRIPE)], dstripe)

            @pl.loop(0, STRIPE // 16)
            def _inv(i):
                v = dstripe[pl.ds(16 * i, 16)]
                dstripe[pl.ds(16 * i, 16)] = 1.0 / (v + 1e-9)
            pltpu.sync_copy(dstripe, dens[r].at[pl.ds(s * i32(STRIPE), STRIPE)])
        plsc.subcore_barrier()

        for r in range(R):
            pltpu.sync_copy(dens[r], invb)

            @pl.loop(0, NCH)
            def _p1c(j, r=r):
                row = j + i32(r * NCH)
                for k in range(8):
                    dv = dstb[row, pl.ds(16 * k, 16)]
                    inv = plsc.load_gather(invb, [dv])
                    exb[row, pl.ds(16 * k, 16)] = exb[row, pl.ds(16 * k, 16)] * inv

        for q in range(NPQ):
            piece = c * i32(NPQ) + i32(q)
            if q > 0:
                plsc.subcore_barrier()
                for k8 in range(STRIPE // WBR):
                    pltpu.sync_copy(
                        wbuf, acc.at[pl.ds(s * i32(STRIPE) + i32(k8 * WBR), WBR)])
            plsc.subcore_barrier()
            for r in range(R):
                if dout == 256:
                    off = (c + i32(2 * r)) * i32(2 * NPAD) + i32(q)
                else:
                    off = (piece + i32(r * NP)) * i32(NPAD)

                @pl.loop(0, NGR)
                def _p2(g, r=r, off=off):
                    descs = []
                    for b in range(NB):
                        row = g * i32(NB) + i32(b) + i32(r * NCH)

                        @pl.when(g > 0)
                        def _dr(b=b):
                            drain_scatter(b)
                        for k in range(8):
                            sv = srcb[row, pl.ds(16 * k, 16)]
                            if dout == 256:
                                idxb[b, pl.ds(16 * k, 16)] = sv + sv + off
                            else:
                                idxb[b, pl.ds(16 * k, 16)] = sv + off
                        descs.append(pltpu.async_copy(
                            zf_hbm.at[idxb.at[b]], gbufs.at[b], gsems[b]))
                    for b in range(NB):
                        row = g * i32(NB) + i32(b) + i32(r * NCH)
                        descs[b].wait()

                        @pl.loop(0, CH, unroll=4)
                        def _scale(e2, b=b, row=row):
                            av = plsc.load_gather(
                                exb.at[row], [jnp.full((16,), e2, jnp.int32)])
                            for k in range(KD):
                                gbufs[b, e2, pl.ds(16 * k, 16)] = (
                                    gbufs[b, e2, pl.ds(16 * k, 16)] * av)
                        pltpu.async_copy(gbufs.at[b], acc.at[dstb.at[row]],
                                         ssems[b], add=True)
                for b in range(NB):
                    drain_scatter(b)
            plsc.subcore_barrier()

            if not add_bias:
                for k8 in range(STRIPE // WBR):
                    base = s * i32(STRIPE) + i32(k8 * WBR)

                    @pl.when(base < i32(N))
                    def _wbr(base=base, piece=piece):
                        pltpu.sync_copy(acc.at[pl.ds(base, WBR)], wbuf)
                        pltpu.sync_copy(wbuf,
                                        out_hbm.at[piece, pl.ds(base, WBR)])
            else:
                pltpu.sync_copy(bsum_hbm.at[piece], bbuf)
                for k8 in range(STRIPE // WBR):
                    base = s * i32(STRIPE) + i32(k8 * WBR)

                    @pl.when(base < i32(N))
                    def _wb(base=base, piece=piece):
                        pltpu.sync_copy(acc.at[pl.ds(base, WBR)], wbuf)

                        @pl.loop(0, WBR)
                        def _row(i):
                            for k in range(KD):
                                v = (wbuf[i, pl.ds(16 * k, 16)]
                                     + bbuf[pl.ds(16 * k, 16)])
                                if relu:
                                    v = jnp.maximum(v, 0.0)
                                wbuf[i, pl.ds(16 * k, 16)] = v
                        pltpu.sync_copy(wbuf,
                                        out_hbm.at[piece, pl.ds(base, WBR)])
            if q + 1 < NPQ:
                @pl.loop(0, WBR)
                def _zw2(i):
                    for k in range(KD):
                        wbuf[i, pl.ds(16 * k, 16)] = zero16

    mesh = plsc.VectorSubcoreMesh(
        core_axis_name="c", subcore_axis_name="s",
        num_cores=NC, num_subcores=NS)
    fn = pl.kernel(
        body,
        out_type=jax.ShapeDtypeStruct((NP, N, PW), f32),
        mesh=mesh,
        scratch_types=[
            pltpu.VMEM((R * NCH, CH), jnp.int32),
            pltpu.VMEM((R * NCH, CH), jnp.int32),
            pltpu.VMEM((R * NCH, CH), f32),
            pltpu.VMEM((NPAD,), f32),
            pltpu.VMEM((NPAD,), f32),
            pltpu.VMEM((NPAD,), f32),
            pltpu.VMEM((STRIPE,), f32),
            pltpu.VMEM((NB, CH), jnp.int32),
            pltpu.VMEM((CH,), f32),
            pltpu.VMEM((NB, CH, PW), f32),
            pltpu.VMEM((WBR, PW), f32),
            pltpu.VMEM((PW,), f32),
            pltpu.VMEM_SHARED((NPAD, PW), f32),
            pltpu.VMEM_SHARED((NPAD,), f32),
            pltpu.VMEM_SHARED((NPAD,), f32),
            pltpu.VMEM_SHARED((NPAD,), f32),
            pltpu.VMEM_SHARED((NPAD,), f32),
        ] + [pltpu.SemaphoreType.DMA] * (2 * NB + 2),
        compiler_params=pltpu.CompilerParams(
            needs_layout_passes=False, use_tc_tiling_on_sc=False),
    )
    return fn(Zf, el, er, src3, dst3, bsum)


def _edge_layout(eis):
    src = jnp.stack([ei[0] for ei in eis]).astype(jnp.int32)
    dst = jnp.stack([ei[1] for ei in eis]).astype(jnp.int32)

    def lay(a):
        a = a.reshape(R, NS, EREAL)
        a = jnp.pad(a, ((0, 0), (0, 0), (0, EPT - EREAL)))
        a = a.reshape(R, NS, NCH, CH).transpose(1, 0, 2, 3)
        return a.reshape(NS * R * NCH, CH)
    return lay(src), lay(dst)


def kernel(x, edge_index0, edge_index1, edge_index2, edge_index3,
           W0, al0, ar0, b0, hb0, W1, al1, ar1, b1, hb1,
           W2, al2, ar2, b2, hb2):
    with jax.enable_x64(False):
        src3, dst3 = _edge_layout(
            [edge_index0, edge_index1, edge_index2, edge_index3])

        h = x
        np_in, pw_in = 1, H
        bias_prev = None
        for li, (W, al, ar, b, hb, act) in enumerate((
                (W0, al0, ar0, b0, hb0, True),
                (W1, al1, ar1, b1, hb1, True),
                (W2, al2, ar2, b2, hb2, False))):
            dout = W.shape[-1]
            NP, PW = _pieces(dout)
            Zf, el, er = _tc_dense(h, W, al, ar, dout, np_in, pw_in, bias_prev)
            bsum = (b.sum(0) + hb).reshape(NP, PW).astype(jnp.float32)
            last = li == 2
            h = _sc_edge(Zf, el, er, src3, dst3, bsum, dout,
                         relu=False, add_bias=last)
            np_in, pw_in = NP, PW
            bias_prev = bsum
        return jnp.concatenate([h[i] for i in range(h.shape[0])], axis=-1)


# speedup vs baseline: 1.1741x; 1.1741x over previous
import jax
import jax.numpy as jnp
from jax import lax
from jax.experimental import pallas as pl
from jax.experimental.pallas import tpu as pltpu
from jax.experimental.pallas import tpu_sc as plsc

N = 10000
NPAD = 10240
H = 256
R = 4
E = 40000
NS = 16
NC = 2
CH = 128
NCH = 20
EPT = CH * NCH
EREAL = E // NS
STRIPE = NPAD // NS
WBR = 80


def _pieces(dout):
    pw = min(64, dout // NC)
    return dout // pw, pw


def _tc_dense(h_parts, W, al, ar, dout, np_in, pw_in, bias_prev):
    NP, PW = _pieces(dout)
    bm = 1024
    MB = NPAD // bm

    def body(h_ref, w_ref, al_ref, ar_ref, *rest):
        if bias_prev is not None:
            b_ref, z_ref, el_ref, er_ref = rest
        else:
            z_ref, el_ref, er_ref = rest
        if np_in == 1:
            hb = h_ref[...]
        else:
            hb = jnp.concatenate([h_ref[i] for i in range(np_in)], axis=-1)
        if bias_prev is not None:
            bfull = jnp.concatenate(
                [b_ref[i] for i in range(np_in)], axis=-1)
            hb = jnp.maximum(hb + bfull, 0.0)
        w = w_ref[0]
        z = jnp.dot(hb, w, preferred_element_type=jnp.float32)
        if dout == 2 * H // 2:
            for ch in range(NC):
                z_ref[ch] = z[:, ch * 128:(ch + 1) * 128]
        else:
            for p in range(NP):
                z_ref[p] = z[:, p * PW:(p + 1) * PW]
        el_ref[0, 0] = jnp.dot(z, al_ref[0, 0], preferred_element_type=jnp.float32)
        er_ref[0, 0] = jnp.dot(z, ar_ref[0, 0], preferred_element_type=jnp.float32)

    if np_in == 1:
        h_spec = pl.BlockSpec((bm, H), lambda m, r: (m, 0))
    else:
        h_spec = pl.BlockSpec((np_in, bm, pw_in), lambda m, r: (0, m, 0))
    in_specs = [
            h_spec,
            pl.BlockSpec((1, H, dout), lambda m, r: (r, 0, 0)),
            pl.BlockSpec((1, 1, dout), lambda m, r: (r, 0, 0)),
            pl.BlockSpec((1, 1, dout), lambda m, r: (r, 0, 0)),
    ]
    args = [h_parts, W, al.reshape(R, 1, dout), ar.reshape(R, 1, dout)]
    if bias_prev is not None:
        in_specs.append(
            pl.BlockSpec((np_in, 1, pw_in), lambda m, r: (0, 0, 0)))
        args.append(bias_prev.reshape(np_in, 1, pw_in))
    zf, el, er = pl.pallas_call(
        body,
        grid=(MB, R),
        in_specs=in_specs,
        out_specs=[
            (pl.BlockSpec((NC, bm, 128), lambda m, r: (r, m, 0))
             if dout == 256 else
             pl.BlockSpec((NP, bm, PW), lambda m, r: (r, m, 0))),
            pl.BlockSpec((1, 1, bm), lambda m, r: (r, 0, m)),
            pl.BlockSpec((1, 1, bm), lambda m, r: (r, 0, m)),
        ],
        out_shape=[
            (jax.ShapeDtypeStruct((R * NC, NPAD, 128), jnp.float32)
             if dout == 256 else
             jax.ShapeDtypeStruct((R * NP, NPAD, PW), jnp.float32)),
            jax.ShapeDtypeStruct((R, 1, NPAD), jnp.float32),
            jax.ShapeDtypeStruct((R, 1, NPAD), jnp.float32),
        ],
    )(*args)
    return (zf.reshape(R * NP * NPAD, PW), el.reshape(R, NPAD),
            er.reshape(R, NPAD))


def _sc_edge(Zf, el, er, src3, dst3, bsum, dout, relu, add_bias):
    NP, PW = _pieces(dout)
    NPQ = NP // NC
    KD = PW // 16
    f32 = jnp.float32

    NB = 2
    NGR = NCH // NB

    def body(zf_hbm, el_hbm, er_hbm, src_hbm, dst_hbm, bsum_hbm, out_hbm,
             srcb, dstb, exb, elb, erb, invb, dstripe, idxb, abuf, gbufs,
             wbuf, bbuf, acc, den0, den1, den2, den3,
             gsem0, gsem1, ssem0, ssem1,
             dsem0, dsem1):
        c = lax.axis_index("c").astype(jnp.int32)
        s = lax.axis_index("s").astype(jnp.int32)
        i32 = jnp.int32
        dens = [den0, den1, den2, den3]
        gsems = [gsem0, gsem1]
        ssems = [ssem0, ssem1]
        dsems = [dsem0, dsem1]
        zero16 = jnp.zeros((16,), f32)

        def drain_scatter(b):
            pltpu.make_async_copy(
                zf_hbm.at[pl.ds(0, CH)], gbufs.at[b], ssems[b]).wait()

        def drain_den(b):
            pltpu.make_async_copy(
                el_hbm.at[0, pl.ds(0, CH)], abuf, dsems[b]).wait()

        pltpu.sync_copy(src_hbm.at[pl.ds(s * i32(R * NCH), R * NCH)], srcb)
        pltpu.sync_copy(dst_hbm.at[pl.ds(s * i32(R * NCH), R * NCH)], dstb)

        @pl.loop(0, WBR)
        def _zw(i):
            for k in range(KD):
                wbuf[i, pl.ds(16 * k, 16)] = zero16
        for k8 in range(STRIPE // WBR):
            pltpu.sync_copy(wbuf, acc.at[pl.ds(s * i32(STRIPE) + i32(k8 * WBR), WBR)])
        for k in range(8):
            abuf[pl.ds(16 * k, 16)] = zero16
        for r in range(R):
            for k5 in range(STRIPE // CH):
                pltpu.sync_copy(abuf, dens[r].at[pl.ds(s * i32(STRIPE) + i32(k5 * CH), CH)])
        plsc.subcore_barrier()

        for r in range(R):
            pltpu.sync_copy(el_hbm.at[r], elb)
            pltpu.sync_copy(er_hbm.at[r], erb)

            @pl.loop(0, NCH // 2)
            def _p1(g, r=r):
                for b in range(2):
                    j = g * i32(2) + i32(b)
                    row = j + i32(r * NCH)
                    for k in range(8):
                        sv = srcb[row, pl.ds(16 * k, 16)]
                        dv = dstb[row, pl.ds(16 * k, 16)]
                        ev = (plsc.load_gather(elb, [sv])
                              + plsc.load_gather(erb, [dv]))
                        ev = jnp.where(ev > 0, ev, 0.2 * ev)
                        ex = jnp.exp(ev)
                        pos = j * i32(CH) + i32(16 * k) + lax.iota(jnp.int32, 16)
                        ex = jnp.where(pos < EREAL, ex, 0.0)
                        exb[row, pl.ds(16 * k, 16)] = ex

                    @pl.when(g > 0)
                    def _dr(b=b):
                        drain_den(b)
                    pltpu.async_copy(exb.at[row], dens[r].at[dstb.at[row]],
                                     dsems[b], add=True)
            for b in range(2):
                drain_den(b)
        plsc.subcore_barrier()

        for r in range(R):
            pltpu.sync_copy(dens[r].at[pl.ds(s * i32(STRIPE), STRIPE)], dstripe)

            @pl.loop(0, STRIPE // 16)
            def _inv(i):
                v = dstripe[pl.ds(16 * i, 16)]
                dstripe[pl.ds(16 * i, 16)] = 1.0 / (v + 1e-9)
            pltpu.sync_copy(dstripe, dens[r].at[pl.ds(s * i32(STRIPE), STRIPE)])
        plsc.subcore_barrier()

        for r in range(R):
            pltpu.sync_copy(dens[r], invb)

            @pl.loop(0, NCH)
            def _p1c(j, r=r):
                row = j + i32(r * NCH)
                for k in range(8):
                    dv = dstb[row, pl.ds(16 * k, 16)]
                    inv = plsc.load_gather(invb, [dv])
                    exb[row, pl.ds(16 * k, 16)] = exb[row, pl.ds(16 * k, 16)] * inv

        for q in range(NPQ):
            piece = c * i32(NPQ) + i32(q)
            if q > 0:
                plsc.subcore_barrier()
                for k8 in range(STRIPE // WBR):
                    pltpu.sync_copy(
                        wbuf, acc.at[pl.ds(s * i32(STRIPE) + i32(k8 * WBR), WBR)])
            plsc.subcore_barrier()
            for r in range(R):
                if dout == 256:
                    off = (c + i32(2 * r)) * i32(2 * NPAD) + i32(q)
                else:
                    off = (piece + i32(r * NP)) * i32(NPAD)

                @pl.loop(0, NGR)
                def _p2(g, r=r, off=off):
                    descs = []
                    for b in range(NB):
                        row = g * i32(NB) + i32(b) + i32(r * NCH)

                        @pl.when(g > 0)
                        def _dr(b=b):
                            drain_scatter(b)
                        for k in range(8):
                            sv = srcb[row, pl.ds(16 * k, 16)]
                            if dout == 256:
                                idxb[b, pl.ds(16 * k, 16)] = sv + sv + off
                            else:
                                idxb[b, pl.ds(16 * k, 16)] = sv + off
                        descs.append(pltpu.async_copy(
                            zf_hbm.at[idxb.at[b]], gbufs.at[b], gsems[b]))
                    for b in range(NB):
                        row = g * i32(NB) + i32(b) + i32(r * NCH)
                        descs[b].wait()

                        @pl.loop(0, CH, unroll=4)
                        def _scale(e2, b=b, row=row):
                            av = plsc.load_gather(
                                exb.at[row], [jnp.full((16,), e2, jnp.int32)])
                            for k in range(KD):
                                gbufs[b, e2, pl.ds(16 * k, 16)] = (
                                    gbufs[b, e2, pl.ds(16 * k, 16)] * av)
                        pltpu.async_copy(gbufs.at[b], acc.at[dstb.at[row]],
                                         ssems[b], add=True)
                for b in range(NB):
                    drain_scatter(b)
            plsc.subcore_barrier()

            if not add_bias:
                for k8 in range(STRIPE // WBR):
                    base = s * i32(STRIPE) + i32(k8 * WBR)

                    @pl.when(base < i32(N))
                    def _wbr(base=base, piece=piece):
                        pltpu.sync_copy(acc.at[pl.ds(base, WBR)], wbuf)
                        pltpu.sync_copy(wbuf,
                                        out_hbm.at[piece, pl.ds(base, WBR)])
            else:
                pltpu.sync_copy(bsum_hbm.at[piece], bbuf)
                for k8 in range(STRIPE // WBR):
                    base = s * i32(STRIPE) + i32(k8 * WBR)

                    @pl.when(base < i32(N))
                    def _wb(base=base, piece=piece):
                        pltpu.sync_copy(acc.at[pl.ds(base, WBR)], wbuf)

                        @pl.loop(0, WBR)
                        def _row(i):
                            for k in range(KD):
                                v = (wbuf[i, pl.ds(16 * k, 16)]
                                     + bbuf[pl.ds(16 * k, 16)])
                                if relu:
                                    v = jnp.maximum(v, 0.0)
                                wbuf[i, pl.ds(16 * k, 16)] = v
                        pltpu.sync_copy(wbuf,
                                        out_hbm.at[piece, pl.ds(base, WBR)])
            if q + 1 < NPQ:
                @pl.loop(0, WBR)
                def _zw2(i):
                    for k in range(KD):
                        wbuf[i, pl.ds(16 * k, 16)] = zero16

    mesh = plsc.VectorSubcoreMesh(
        core_axis_name="c", subcore_axis_name="s",
        num_cores=NC, num_subcores=NS)
    fn = pl.kernel(
        body,
        out_type=jax.ShapeDtypeStruct((NP, N, PW), f32),
        mesh=mesh,
        scratch_types=[
            pltpu.VMEM((R * NCH, CH), jnp.int32),
            pltpu.VMEM((R * NCH, CH), jnp.int32),
            pltpu.VMEM((R * NCH, CH), f32),
            pltpu.VMEM((NPAD,), f32),
            pltpu.VMEM((NPAD,), f32),
            pltpu.VMEM((NPAD,), f32),
            pltpu.VMEM((STRIPE,), f32),
            pltpu.VMEM((NB, CH), jnp.int32),
            pltpu.VMEM((CH,), f32),
            pltpu.VMEM((NB, CH, PW), f32),
            pltpu.VMEM((WBR, PW), f32),
            pltpu.VMEM((PW,), f32),
            pltpu.VMEM_SHARED((NPAD, PW), f32),
            pltpu.VMEM_SHARED((NPAD,), f32),
            pltpu.VMEM_SHARED((NPAD,), f32),
            pltpu.VMEM_SHARED((NPAD,), f32),
            pltpu.VMEM_SHARED((NPAD,), f32),
        ] + [pltpu.SemaphoreType.DMA] * (2 * NB + 2),
        compiler_params=pltpu.CompilerParams(
            needs_layout_passes=False, use_tc_tiling_on_sc=False),
    )
    return fn(Zf, el, er, src3, dst3, bsum)


def _edge_layout(eis):
    src = jnp.stack([ei[0] for ei in eis]).astype(jnp.int32)
    dst = jnp.stack([ei[1] for ei in eis]).astype(jnp.int32)

    def lay(a):
        a = a.reshape(R, NS, EREAL)
        a = jnp.pad(a, ((0, 0), (0, 0), (0, EPT - EREAL)))
        a = a.reshape(R, NS, NCH, CH).transpose(1, 0, 2, 3)
        return a.reshape(NS * R * NCH, CH)
    return lay(src), lay(dst)


def kernel(x, edge_index0, edge_index1, edge_index2, edge_index3,
           W0, al0, ar0, b0, hb0, W1, al1, ar1, b1, hb1,
           W2, al2, ar2, b2, hb2):
    with jax.enable_x64(False):
        src3, dst3 = _edge_layout(
            [edge_index0, edge_index1, edge_index2, edge_index3])

        h = x
        np_in, pw_in = 1, H
        bias_prev = None
        for li, (W, al, ar, b, hb, act) in enumerate((
                (W0, al0, ar0, b0, hb0, True),
                (W1, al1, ar1, b1, hb1, True),
                (W2, al2, ar2, b2, hb2, False))):
            dout = W.shape[-1]
            NP, PW = _pieces(dout)
            Zf, el, er = _tc_dense(h, W, al, ar, dout, np_in, pw_in, bias_prev)
            bsum = (b.sum(0) + hb).reshape(NP, PW).astype(jnp.float32)
            last = li == 2
            h = _sc_edge(Zf, el, er, src3, dst3, bsum, dout,
                         relu=False, add_bias=last)
            np_in, pw_in = NP, PW
            bias_prev = bsum
        return jnp.concatenate([h[i] for i in range(h.shape[0])], axis=-1)


# speedup vs baseline: 1.1775x; 1.0029x over previous
import jax
import jax.numpy as jnp
from jax import lax
from jax.experimental import pallas as pl
from jax.experimental.pallas import tpu as pltpu
from jax.experimental.pallas import tpu_sc as plsc

N = 10000
NPAD = 10240
H = 256
R = 4
E = 40000
NS = 16
NC = 2
CH = 128
NCH = 20
EPT = CH * NCH
EREAL = E // NS
STRIPE = NPAD // NS
WBR = 80


def _pieces(dout):
    pw = min(64, dout // NC)
    return dout // pw, pw


def _tc_dense(h_parts, W, al, ar, dout, np_in, pw_in, bias_prev):
    NP, PW = _pieces(dout)
    bm = 1024
    MB = NPAD // bm

    def body(h_ref, w_ref, al_ref, ar_ref, *rest):
        if bias_prev is not None:
            b_ref, z_ref, el_ref, er_ref = rest
        else:
            z_ref, el_ref, er_ref = rest
        if np_in == 1:
            hb = h_ref[...]
        else:
            hb = jnp.concatenate([h_ref[i] for i in range(np_in)], axis=-1)
        if bias_prev is not None:
            bfull = jnp.concatenate(
                [b_ref[i] for i in range(np_in)], axis=-1)
            hb = jnp.maximum(hb + bfull, 0.0)
        w = w_ref[0]
        z = jnp.dot(hb, w, preferred_element_type=jnp.float32)
        if dout == 2 * H // 2:
            for ch in range(NC):
                z_ref[ch] = z[:, ch * 128:(ch + 1) * 128]
        else:
            for p in range(NP):
                z_ref[p] = z[:, p * PW:(p + 1) * PW]
        el_ref[0, 0] = jnp.dot(z, al_ref[0, 0], preferred_element_type=jnp.float32)
        er_ref[0, 0] = jnp.dot(z, ar_ref[0, 0], preferred_element_type=jnp.float32)

    if np_in == 1:
        h_spec = pl.BlockSpec((bm, H), lambda m, r: (m, 0))
    else:
        h_spec = pl.BlockSpec((np_in, bm, pw_in), lambda m, r: (0, m, 0))
    in_specs = [
            h_spec,
            pl.BlockSpec((1, H, dout), lambda m, r: (r, 0, 0)),
            pl.BlockSpec((1, 1, dout), lambda m, r: (r, 0, 0)),
            pl.BlockSpec((1, 1, dout), lambda m, r: (r, 0, 0)),
    ]
    args = [h_parts, W, al.reshape(R, 1, dout), ar.reshape(R, 1, dout)]
    if bias_prev is not None:
        in_specs.append(
            pl.BlockSpec((np_in, 1, pw_in), lambda m, r: (0, 0, 0)))
        args.append(bias_prev.reshape(np_in, 1, pw_in))
    zf, el, er = pl.pallas_call(
        body,
        grid=(MB, R),
        in_specs=in_specs,
        out_specs=[
            (pl.BlockSpec((NC, bm, 128), lambda m, r: (r, m, 0))
             if dout == 256 else
             pl.BlockSpec((NP, bm, PW), lambda m, r: (r, m, 0))),
            pl.BlockSpec((1, 1, bm), lambda m, r: (r, 0, m)),
            pl.BlockSpec((1, 1, bm), lambda m, r: (r, 0, m)),
        ],
        out_shape=[
            (jax.ShapeDtypeStruct((R * NC, NPAD, 128), jnp.float32)
             if dout == 256 else
             jax.ShapeDtypeStruct((R * NP, NPAD, PW), jnp.float32)),
            jax.ShapeDtypeStruct((R, 1, NPAD), jnp.float32),
            jax.ShapeDtypeStruct((R, 1, NPAD), jnp.float32),
        ],
    )(*args)
    return (zf.reshape(R * NP * NPAD, PW), el.reshape(R, NPAD),
            er.reshape(R, NPAD))


def _sc_edge(Zf, el, er, src3, dst3, bsum, dout, relu, add_bias):
    NP, PW = _pieces(dout)
    NPQ = NP // NC
    KD = PW // 16
    f32 = jnp.float32

    NB = 2
    NGR = NCH // NB

    def body(zf_hbm, el_hbm, er_hbm, src_hbm, dst_hbm, bsum_hbm, out_hbm,
             srcb, dstb, exb, elb, erb, invb, dstripe, idxb, abuf, gbufs,
             wbuf, bbuf, acc, den0, den1, den2, den3,
             gsem0, gsem1, ssem0, ssem1,
             dsem0, dsem1):
        c = lax.axis_index("c").astype(jnp.int32)
        s = lax.axis_index("s").astype(jnp.int32)
        i32 = jnp.int32
        dens = [den0, den1, den2, den3]
        gsems = [gsem0, gsem1]
        ssems = [ssem0, ssem1]
        dsems = [dsem0, dsem1]
        zero16 = jnp.zeros((16,), f32)

        def drain_scatter(b):
            pltpu.make_async_copy(
                zf_hbm.at[pl.ds(0, CH)], gbufs.at[b], ssems[b]).wait()

        def drain_den(b):
            pltpu.make_async_copy(
                el_hbm.at[0, pl.ds(0, CH)], abuf, dsems[b]).wait()

        pltpu.sync_copy(src_hbm.at[pl.ds(s * i32(R * NCH), R * NCH)], srcb)
        pltpu.sync_copy(dst_hbm.at[pl.ds(s * i32(R * NCH), R * NCH)], dstb)

        @pl.loop(0, WBR)
        def _zw(i):
            for k in range(KD):
                wbuf[i, pl.ds(16 * k, 16)] = zero16
        zero_descs = [
            pltpu.async_copy(
                wbuf, acc.at[pl.ds(s * i32(STRIPE) + i32(k8 * WBR), WBR)],
                gsems[k8 % 2])
            for k8 in range(STRIPE // WBR)]
        for k in range(8):
            abuf[pl.ds(16 * k, 16)] = zero16
        for r in range(R):
            for k5 in range(STRIPE // CH):
                pltpu.sync_copy(abuf, dens[r].at[pl.ds(s * i32(STRIPE) + i32(k5 * CH), CH)])
        plsc.subcore_barrier()

        for r in range(R):
            pltpu.sync_copy(el_hbm.at[r], elb)
            pltpu.sync_copy(er_hbm.at[r], erb)

            @pl.loop(0, NCH // 2)
            def _p1(g, r=r):
                for b in range(2):
                    j = g * i32(2) + i32(b)
                    row = j + i32(r * NCH)
                    for k in range(8):
                        sv = srcb[row, pl.ds(16 * k, 16)]
                        dv = dstb[row, pl.ds(16 * k, 16)]
                        ev = (plsc.load_gather(elb, [sv])
                              + plsc.load_gather(erb, [dv]))
                        ev = jnp.where(ev > 0, ev, 0.2 * ev)
                        ex = jnp.exp(ev)
                        pos = j * i32(CH) + i32(16 * k) + lax.iota(jnp.int32, 16)
                        ex = jnp.where(pos < EREAL, ex, 0.0)
                        exb[row, pl.ds(16 * k, 16)] = ex

                    @pl.when(g > 0)
                    def _dr(b=b):
                        drain_den(b)
                    pltpu.async_copy(exb.at[row], dens[r].at[dstb.at[row]],
                                     dsems[b], add=True)
            for b in range(2):
                drain_den(b)
        plsc.subcore_barrier()

        for r in range(R):
            pltpu.sync_copy(dens[r].at[pl.ds(s * i32(STRIPE), STRIPE)], dstripe)

            @pl.loop(0, STRIPE // 16)
            def _inv(i):
                v = dstripe[pl.ds(16 * i, 16)]
                dstripe[pl.ds(16 * i, 16)] = 1.0 / (v + 1e-9)
            pltpu.sync_copy(dstripe, dens[r].at[pl.ds(s * i32(STRIPE), STRIPE)])
        plsc.subcore_barrier()

        for r in range(R):
            pltpu.sync_copy(dens[r], invb)

            @pl.loop(0, NCH)
            def _p1c(j, r=r):
                row = j + i32(r * NCH)
                for k in range(8):
                    dv = dstb[row, pl.ds(16 * k, 16)]
                    inv = plsc.load_gather(invb, [dv])
                    exb[row, pl.ds(16 * k, 16)] = exb[row, pl.ds(16 * k, 16)] * inv

        for d in zero_descs:
            d.wait()

        for q in range(NPQ):
            piece = c * i32(NPQ) + i32(q)
            if q > 0:
                plsc.subcore_barrier()
                for k8 in range(STRIPE // WBR):
                    pltpu.sync_copy(
                        wbuf, acc.at[pl.ds(s * i32(STRIPE) + i32(k8 * WBR), WBR)])
            plsc.subcore_barrier()
            for r in range(R):
                if dout == 256:
                    off = (c + i32(2 * r)) * i32(2 * NPAD) + i32(q)
                else:
                    off = (piece + i32(r * NP)) * i32(NPAD)

                @pl.loop(0, NGR)
                def _p2(g, r=r, off=off):
                    descs = []
                    for b in range(NB):
                        row = g * i32(NB) + i32(b) + i32(r * NCH)

                        @pl.when(g > 0)
                        def _dr(b=b):
                            drain_scatter(b)
                        for k in range(8):
                            sv = srcb[row, pl.ds(16 * k, 16)]
                            if dout == 256:
                                idxb[b, pl.ds(16 * k, 16)] = sv + sv + off
                            else:
                                idxb[b, pl.ds(16 * k, 16)] = sv + off
                        descs.append(pltpu.async_copy(
                            zf_hbm.at[idxb.at[b]], gbufs.at[b], gsems[b]))
                    for b in range(NB):
                        row = g * i32(NB) + i32(b) + i32(r * NCH)
                        descs[b].wait()

                        @pl.loop(0, CH, unroll=4)
                        def _scale(e2, b=b, row=row):
                            av = plsc.load_gather(
                                exb.at[row], [jnp.full((16,), e2, jnp.int32)])
                            for k in range(KD):
                                gbufs[b, e2, pl.ds(16 * k, 16)] = (
                                    gbufs[b, e2, pl.ds(16 * k, 16)] * av)
                        pltpu.async_copy(gbufs.at[b], acc.at[dstb.at[row]],
                                         ssems[b], add=True)
                for b in range(NB):
                    drain_scatter(b)
            plsc.subcore_barrier()

            if not add_bias:
                for k8 in range(STRIPE // WBR):
                    base = s * i32(STRIPE) + i32(k8 * WBR)

                    @pl.when(base < i32(N))
                    def _wbr(base=base, piece=piece):
                        pltpu.sync_copy(acc.at[pl.ds(base, WBR)], wbuf)
                        pltpu.sync_copy(wbuf,
                                        out_hbm.at[piece, pl.ds(base, WBR)])
            else:
                pltpu.sync_copy(bsum_hbm.at[piece], bbuf)
                for k8 in range(STRIPE // WBR):
                    base = s * i32(STRIPE) + i32(k8 * WBR)

                    @pl.when(base < i32(N))
                    def _wb(base=base, piece=piece):
                        pltpu.sync_copy(acc.at[pl.ds(base, WBR)], wbuf)

                        @pl.loop(0, WBR)
                        def _row(i):
                            for k in range(KD):
                                v = (wbuf[i, pl.ds(16 * k, 16)]
                                     + bbuf[pl.ds(16 * k, 16)])
                                if relu:
                                    v = jnp.maximum(v, 0.0)
                                wbuf[i, pl.ds(16 * k, 16)] = v
                        pltpu.sync_copy(wbuf,
                                        out_hbm.at[piece, pl.ds(base, WBR)])
            if q + 1 < NPQ:
                @pl.loop(0, WBR)
                def _zw2(i):
                    for k in range(KD):
                        wbuf[i, pl.ds(16 * k, 16)] = zero16

    mesh = plsc.VectorSubcoreMesh(
        core_axis_name="c", subcore_axis_name="s",
        num_cores=NC, num_subcores=NS)
    fn = pl.kernel(
        body,
        out_type=jax.ShapeDtypeStruct((NP, N, PW), f32),
        mesh=mesh,
        scratch_types=[
            pltpu.VMEM((R * NCH, CH), jnp.int32),
            pltpu.VMEM((R * NCH, CH), jnp.int32),
            pltpu.VMEM((R * NCH, CH), f32),
            pltpu.VMEM((NPAD,), f32),
            pltpu.VMEM((NPAD,), f32),
            pltpu.VMEM((NPAD,), f32),
            pltpu.VMEM((STRIPE,), f32),
            pltpu.VMEM((NB, CH), jnp.int32),
            pltpu.VMEM((CH,), f32),
            pltpu.VMEM((NB, CH, PW), f32),
            pltpu.VMEM((WBR, PW), f32),
            pltpu.VMEM((PW,), f32),
            pltpu.VMEM_SHARED((NPAD, PW), f32),
            pltpu.VMEM_SHARED((NPAD,), f32),
            pltpu.VMEM_SHARED((NPAD,), f32),
            pltpu.VMEM_SHARED((NPAD,), f32),
            pltpu.VMEM_SHARED((NPAD,), f32),
        ] + [pltpu.SemaphoreType.DMA] * (2 * NB + 2),
        compiler_params=pltpu.CompilerParams(
            needs_layout_passes=False, use_tc_tiling_on_sc=False),
    )
    return fn(Zf, el, er, src3, dst3, bsum)


def _edge_layout(eis):
    src = jnp.stack([ei[0] for ei in eis]).astype(jnp.int32)
    dst = jnp.stack([ei[1] for ei in eis]).astype(jnp.int32)

    def lay(a):
        a = a.reshape(R, NS, EREAL)
        a = jnp.pad(a, ((0, 0), (0, 0), (0, EPT - EREAL)))
        a = a.reshape(R, NS, NCH, CH).transpose(1, 0, 2, 3)
        return a.reshape(NS * R * NCH, CH)
    return lay(src), lay(dst)


def kernel(x, edge_index0, edge_index1, edge_index2, edge_index3,
           W0, al0, ar0, b0, hb0, W1, al1, ar1, b1, hb1,
           W2, al2, ar2, b2, hb2):
    with jax.enable_x64(False):
        src3, dst3 = _edge_layout(
            [edge_index0, edge_index1, edge_index2, edge_index3])

        h = x
        np_in, pw_in = 1, H
        bias_prev = None
        for li, (W, al, ar, b, hb, act) in enumerate((
                (W0, al0, ar0, b0, hb0, True),
                (W1, al1, ar1, b1, hb1, True),
                (W2, al2, ar2, b2, hb2, False))):
            dout = W.shape[-1]
            NP, PW = _pieces(dout)
            Zf, el, er = _tc_dense(h, W, al, ar, dout, np_in, pw_in, bias_prev)
            bsum = (b.sum(0) + hb).reshape(NP, PW).astype(jnp.float32)
            last = li == 2
            h = _sc_edge(Zf, el, er, src3, dst3, bsum, dout,
                         relu=False, add_bias=last)
            np_in, pw_in = NP, PW
            bias_prev = bsum
        return jnp.concatenate([h[i] for i in range(h.shape[0])], axis=-1)
